# baseline (speedup 1.0000x reference)
import functools

import jax
import jax.numpy as jnp
from jax import lax
from jax.experimental import pallas as pl
from jax.experimental.pallas import tpu as pltpu
from jax.experimental.pallas import tpu_sc as plsc

NUM_NODES = 1000000
NUM_RELATIONS = 1000
HIDDEN = 64
BATCH = 16384

NUM_CORES = 2
NUM_SUBCORES = 16
LANES = 16
NW = NUM_CORES * NUM_SUBCORES
BPW = BATCH // NW
CHUNK = 128
NCHUNK = BPW // CHUNK
GROUPS = BPW // LANES


def _newton_rsqrt(x):
    i = plsc.bitcast(x, jnp.int32)
    i = jnp.int32(0x5F3759DF) - (i >> 1)
    y = plsc.bitcast(i, jnp.float32)
    for _ in range(3):
        y = y * (1.5 - 0.5 * x * y * y)
    return y


def _body(head_hbm, rel_hbm, tail_hbm, ent_hbm, relemb_hbm, out_hbm,
          idx_h, idx_r, idx_t, rows_h, rows_r, rows_t, out_v, sem):
    wid = lax.axis_index("s") * NUM_CORES + lax.axis_index("c")
    base = wid * BPW

    for j in range(NCHUNK):
        pltpu.sync_copy(head_hbm.at[pl.ds(base + j * CHUNK, CHUNK)], idx_h.at[j])
        pltpu.sync_copy(rel_hbm.at[pl.ds(base + j * CHUNK, CHUNK)], idx_r.at[j])
        pltpu.sync_copy(tail_hbm.at[pl.ds(base + j * CHUNK, CHUNK)], idx_t.at[j])

    copies = []
    for j in range(NCHUNK):
        dst = pl.ds(j * CHUNK, CHUNK)
        copies.append(pltpu.async_copy(ent_hbm.at[idx_h.at[j]], rows_h.at[dst], sem))
        copies.append(pltpu.async_copy(relemb_hbm.at[idx_r.at[j]], rows_r.at[dst], sem))
        copies.append(pltpu.async_copy(ent_hbm.at[idx_t.at[j]], rows_t.at[dst], sem))
    for c in copies:
        c.wait()

    lane = lax.iota(jnp.int32, LANES)
    zero = jnp.zeros((LANES,), jnp.float32)

    def group(g, carry):
        hh = zero; tt = zero; rr = zero
        hr = zero; ht = zero; rt = zero
        for j in range(LANES):
            i = g * LANES + j
            h = [rows_h[i, pl.ds(c * LANES, LANES)] for c in range(HIDDEN // LANES)]
            r = [rows_r[i, pl.ds(c * LANES, LANES)] for c in range(HIDDEN // LANES)]
            t = [rows_t[i, pl.ds(c * LANES, LANES)] for c in range(HIDDEN // LANES)]

            def dot(a, b):
                v = a[0] * b[0]
                for c in range(1, HIDDEN // LANES):
                    v = v + a[c] * b[c]
                return jnp.sum(v)

            sel = lane == j
            hh = jnp.where(sel, dot(h, h), hh)
            tt = jnp.where(sel, dot(t, t), tt)
            rr = jnp.where(sel, dot(r, r), rr)
            hr = jnp.where(sel, dot(h, r), hr)
            ht = jnp.where(sel, dot(h, t), ht)
            rt = jnp.where(sel, dot(r, t), rt)
        inv_h = _newton_rsqrt(jnp.maximum(hh, 1e-24))
        inv_t = _newton_rsqrt(jnp.maximum(tt, 1e-24))
        d2 = (rr + hh * inv_h * inv_h + tt * inv_t * inv_t
              + 2.0 * (hr * inv_h - ht * (inv_h * inv_t) - rt * inv_t))
        d2 = jnp.maximum(d2, 0.0)
        out_v[pl.ds(g * LANES, LANES)] = d2 * _newton_rsqrt(jnp.maximum(d2, 1e-24))
        return carry

    lax.fori_loop(0, GROUPS, group, None)
    pltpu.sync_copy(out_v, out_hbm.at[pl.ds(base, BPW)])


def _transe_sc(head_index, rel_index, tail_index, ent_emb, rel_emb):
    mesh = plsc.VectorSubcoreMesh(core_axis_name="c", subcore_axis_name="s")
    f = pl.kernel(
        _body,
        out_type=jax.ShapeDtypeStruct((BATCH,), jnp.float32),
        mesh=mesh,
        scratch_types=[
            pltpu.VMEM((NCHUNK, CHUNK), jnp.int32),
            pltpu.VMEM((NCHUNK, CHUNK), jnp.int32),
            pltpu.VMEM((NCHUNK, CHUNK), jnp.int32),
            pltpu.VMEM((BPW, HIDDEN), jnp.float32),
            pltpu.VMEM((BPW, HIDDEN), jnp.float32),
            pltpu.VMEM((BPW, HIDDEN), jnp.float32),
            pltpu.VMEM((BPW,), jnp.float32),
            pltpu.SemaphoreType.DMA,
        ],
        compiler_params=pltpu.CompilerParams(
            needs_layout_passes=False, use_tc_tiling_on_sc=False),
        name="transe_sc",
    )
    return f(head_index, rel_index, tail_index, ent_emb, rel_emb)


def kernel(head_index, rel_index, tail_index, ent_emb, rel_emb):
    return _transe_sc(head_index.astype(jnp.int32), rel_index.astype(jnp.int32),
                      tail_index.astype(jnp.int32), ent_emb, rel_emb)
